# Initial kernel scaffold; baseline (speedup 1.0000x reference)
import functools

import jax
import jax.numpy as jnp
from jax import lax
from jax.experimental import pallas as pl
from jax.experimental.pallas import tpu as pltpu
from jax.experimental.pallas import tpu_sc as plsc

N = 10000
NP = 10240
E = 320000
HEADS = 4
F = 128
D_IN = 128

NC = 2
NS = 16
L = 16

EPT = E // NS
CH = 128
NCHUNK = (EPT + CH - 1) // CH
EPT_PAD = NCHUNK * CH
ROWS_PER_TILE = NP // NS

TCB = 1280
GRID = NP // TCB


def _proj_body(x_ref, w_ref, al_ref, ar_ref, feat_ref, el_ref, er_ref):
    y = jnp.dot(x_ref[...], w_ref[...], preferred_element_type=jnp.float32)
    for h in range(HEADS):
        fh = y[:, h * F:(h + 1) * F]
        feat_ref[h] = fh
        el_ref[h] = jnp.sum(fh * al_ref[h][None, :], axis=-1)
        er_ref[h] = jnp.sum(fh * ar_ref[h][None, :], axis=-1)


def _dense1(x, w, al, ar):
    return pl.pallas_call(
        _proj_body,
        grid=(GRID,),
        in_specs=[
            pl.BlockSpec((TCB, D_IN), lambda i: (i, 0)),
            pl.BlockSpec((D_IN, HEADS * F), lambda i: (0, 0)),
            pl.BlockSpec((HEADS, F), lambda i: (0, 0)),
            pl.BlockSpec((HEADS, F), lambda i: (0, 0)),
        ],
        out_specs=[
            pl.BlockSpec((HEADS, TCB, F), lambda i: (0, i, 0)),
            pl.BlockSpec((HEADS, TCB), lambda i: (0, i)),
            pl.BlockSpec((HEADS, TCB), lambda i: (0, i)),
        ],
        out_shape=[
            jax.ShapeDtypeStruct((HEADS, NP, F), jnp.float32),
            jax.ShapeDtypeStruct((HEADS, NP), jnp.float32),
            jax.ShapeDtypeStruct((HEADS, NP), jnp.float32),
        ],
    )(x, w, al, ar)


def _mid_body(o_ref, b_ref, w_ref, al_ref, ar_ref, feat_ref, el_ref, er_ref):
    acc = jnp.maximum(o_ref[0] + b_ref[0][None, :], 0.0)
    for h in range(1, HEADS):
        acc = acc + jnp.maximum(o_ref[h] + b_ref[h][None, :], 0.0)
    hm = acc * (1.0 / HEADS)
    y = jnp.dot(hm, w_ref[...], preferred_element_type=jnp.float32)
    for h in range(HEADS):
        fh = y[:, h * F:(h + 1) * F]
        feat_ref[h] = fh
        el_ref[h] = jnp.sum(fh * al_ref[h][None, :], axis=-1)
        er_ref[h] = jnp.sum(fh * ar_ref[h][None, :], axis=-1)


def _dense2(o1, b1r, w, al, ar):
    return pl.pallas_call(
        _mid_body,
        grid=(GRID,),
        in_specs=[
            pl.BlockSpec((HEADS, TCB, F), lambda i: (0, i, 0)),
            pl.BlockSpec((HEADS, F), lambda i: (0, 0)),
            pl.BlockSpec((F, HEADS * F), lambda i: (0, 0)),
            pl.BlockSpec((HEADS, F), lambda i: (0, 0)),
            pl.BlockSpec((HEADS, F), lambda i: (0, 0)),
        ],
        out_specs=[
            pl.BlockSpec((HEADS, TCB, F), lambda i: (0, i, 0)),
            pl.BlockSpec((HEADS, TCB), lambda i: (0, i)),
            pl.BlockSpec((HEADS, TCB), lambda i: (0, i)),
        ],
        out_shape=[
            jax.ShapeDtypeStruct((HEADS, NP, F), jnp.float32),
            jax.ShapeDtypeStruct((HEADS, NP), jnp.float32),
            jax.ShapeDtypeStruct((HEADS, NP), jnp.float32),
        ],
    )(o1, b1r, w, al, ar)


def _final_body(o_ref, b_ref, x_ref, out_ref):
    acc = o_ref[0] + b_ref[0][None, :]
    for h in range(1, HEADS):
        acc = acc + (o_ref[h] + b_ref[h][None, :])
    out_ref[...] = acc * (1.0 / HEADS) + x_ref[...]


def _dense3(o2, b2r, x):
    return pl.pallas_call(
        _final_body,
        grid=(GRID,),
        in_specs=[
            pl.BlockSpec((HEADS, TCB, F), lambda i: (0, i, 0)),
            pl.BlockSpec((HEADS, F), lambda i: (0, 0)),
            pl.BlockSpec((TCB, F), lambda i: (i, 0)),
        ],
        out_specs=pl.BlockSpec((TCB, F), lambda i: (i, 0)),
        out_shape=jax.ShapeDtypeStruct((NP, F), jnp.float32),
    )(o2, b2r, x)


_LANE = jnp.arange(L, dtype=jnp.int32)


def _sc_body(src_hbm, dst_hbm, feat_hbm, el_hbm, er_hbm, out_hbm,
             src_v, dst_v, tab_v, tab2_v, e_v, idx_v, rows_v, mx_v,
             allmx_v, z_v, out_sh, den_sh, mx_sh):
    c = lax.axis_index("c")
    s = lax.axis_index("s")

    pltpu.sync_copy(src_hbm.at[s], src_v)
    pltpu.sync_copy(dst_hbm.at[s], dst_v)

    def zrow(i, _):
        for k in range(F // L):
            rows_v[i, pl.ds(k * L, L)] = jnp.zeros((L,), jnp.float32)
        return 0
    lax.fori_loop(0, CH, zrow, 0)

    def zvec(i, _):
        z_v[pl.ds(i * L, L)] = jnp.zeros((L,), jnp.float32)
        return 0
    lax.fori_loop(0, ROWS_PER_TILE // L, zvec, 0)

    for hh in range(HEADS // NC):
        h = c * (HEADS // NC) + hh
        base = h * NP

        pltpu.sync_copy(z_v, den_sh.at[pl.ds(s * ROWS_PER_TILE, ROWS_PER_TILE)])
        for k in range(ROWS_PER_TILE // CH):
            pltpu.sync_copy(
                rows_v, out_sh.at[pl.ds(s * ROWS_PER_TILE + k * CH, CH)])

        pltpu.sync_copy(el_hbm.at[pl.ds(base, NP)], tab_v)
        pltpu.sync_copy(er_hbm.at[pl.ds(base, NP)], tab2_v)

        def chunkA(j, mx):
            for k in range(CH // L):
                sv = src_v[j, pl.ds(k * L, L)]
                dv = dst_v[j, pl.ds(k * L, L)]
                ev = plsc.load_gather(tab_v, [sv]) + plsc.load_gather(tab2_v, [dv])
                ev = jnp.where(ev >= 0.0, ev, 0.2 * ev)
                e_v[j, pl.ds(k * L, L)] = ev
                gid = j * CH + k * L + _LANE
                mx = jnp.maximum(mx, jnp.where(gid < EPT, ev, -1e30))
            return mx
        mx = lax.fori_loop(0, NCHUNK, chunkA, jnp.full((L,), -1e30, jnp.float32))
        mx_v[...] = mx
        pltpu.sync_copy(mx_v, mx_sh.at[s])
        plsc.subcore_barrier()

        pltpu.sync_copy(mx_sh, allmx_v)
        mm = allmx_v[0]
        for r in range(1, NS):
            mm = jnp.maximum(mm, allmx_v[r])
        g = jnp.full((L,), jnp.max(mm), jnp.float32)

        def chunkA2(j, _):
            for k in range(CH // L):
                ev = e_v[j, pl.ds(k * L, L)]
                ex = jnp.exp(ev - g)
                gid = j * CH + k * L + _LANE
                e_v[j, pl.ds(k * L, L)] = jnp.where(gid < EPT, ex, 0.0)
            pltpu.sync_copy(e_v.at[j], den_sh.at[dst_v.at[j]], add=True)
            return 0
        lax.fori_loop(0, NCHUNK, chunkA2, 0)
        plsc.subcore_barrier()

        pltpu.sync_copy(den_sh, tab_v)

        def invv(i, _):
            d = tab_v[pl.ds(i * L, L)]
            tab_v[pl.ds(i * L, L)] = 1.0 / jnp.maximum(d, 1e-9)
            return 0
        lax.fori_loop(0, NP // L, invv, 0)

        def chunkB(j, _):
            for k in range(CH // L):
                dv = dst_v[j, pl.ds(k * L, L)]
                ex = e_v[j, pl.ds(k * L, L)]
                e_v[j, pl.ds(k * L, L)] = ex * plsc.load_gather(tab_v, [dv])
                idx_v[pl.ds(k * L, L)] = src_v[j, pl.ds(k * L, L)] + base
            pltpu.sync_copy(feat_hbm.at[idx_v], rows_v)

            def scale(e, _):
                a = plsc.load_gather(
                    e_v, [jnp.full((L,), j, jnp.int32),
                          jnp.full((L,), e, jnp.int32)])
                for k in range(F // L):
                    rows_v[e, pl.ds(k * L, L)] = rows_v[e, pl.ds(k * L, L)] * a
                return 0
            lax.fori_loop(0, CH, scale, 0)
            pltpu.sync_copy(rows_v, out_sh.at[dst_v.at[j]], add=True)
            return 0
        lax.fori_loop(0, NCHUNK, chunkB, 0)
        plsc.subcore_barrier()

        pltpu.sync_copy(
            out_sh.at[pl.ds(s * ROWS_PER_TILE, ROWS_PER_TILE)],
            out_hbm.at[pl.ds(base + s * ROWS_PER_TILE, ROWS_PER_TILE)])
        plsc.subcore_barrier()

        if hh == 0:
            lax.fori_loop(0, CH, zrow, 0)


_sc_layer = functools.partial(
    pl.kernel,
    out_type=jax.ShapeDtypeStruct((HEADS * NP, F), jnp.float32),
    mesh=plsc.VectorSubcoreMesh(
        core_axis_name="c", subcore_axis_name="s",
        num_cores=NC, num_subcores=NS),
    scratch_types=[
        pltpu.VMEM((NCHUNK, CH), jnp.int32),
        pltpu.VMEM((NCHUNK, CH), jnp.int32),
        pltpu.VMEM((NP,), jnp.float32),
        pltpu.VMEM((NP,), jnp.float32),
        pltpu.VMEM((NCHUNK, CH), jnp.float32),
        pltpu.VMEM((CH,), jnp.int32),
        pltpu.VMEM((CH, F), jnp.float32),
        pltpu.VMEM((L,), jnp.float32),
        pltpu.VMEM((NS, L), jnp.float32),
        pltpu.VMEM((ROWS_PER_TILE,), jnp.float32),
        pltpu.VMEM_SHARED((NP, F), jnp.float32),
        pltpu.VMEM_SHARED((NP,), jnp.float32),
        pltpu.VMEM_SHARED((NS, L), jnp.float32),
    ],
)(_sc_body)


@jax.jit
def _run(in_feat, src3, dst3, W1, al1, ar1, b1, W2, al2, ar2, b2):
    x = jnp.pad(in_feat, ((0, NP - N), (0, 0)))
    feat1, el1, er1 = _dense1(x, W1, al1, ar1)
    o1 = _sc_layer(src3, dst3,
                   feat1.reshape(HEADS * NP, F),
                   el1.reshape(HEADS * NP), er1.reshape(HEADS * NP))
    feat2, el2, er2 = _dense2(o1.reshape(HEADS, NP, F),
                              b1.reshape(HEADS, F), W2, al2, ar2)
    o2 = _sc_layer(src3, dst3,
                   feat2.reshape(HEADS * NP, F),
                   el2.reshape(HEADS * NP), er2.reshape(HEADS * NP))
    out = _dense3(o2.reshape(HEADS, NP, F), b2.reshape(HEADS, F), x)
    return out[:N]


def kernel(in_feat, edge_index, W1, al1, ar1, b1, W2, al2, ar2, b2):
    src = edge_index[0].astype(jnp.int32).reshape(NS, EPT)
    dst = edge_index[1].astype(jnp.int32).reshape(NS, EPT)
    src3 = jnp.pad(src, ((0, 0), (0, EPT_PAD - EPT))).reshape(NS, NCHUNK, CH)
    dst3 = jnp.pad(dst, ((0, 0), (0, EPT_PAD - EPT))).reshape(NS, NCHUNK, CH)
    return _run(in_feat, src3, dst3, W1, al1, ar1, b1, W2, al2, ar2, b2)


# speedup vs baseline: 14.3108x; 14.3108x over previous
import functools

import jax
import jax.numpy as jnp
from jax import lax
from jax.experimental import pallas as pl
from jax.experimental.pallas import tpu as pltpu
from jax.experimental.pallas import tpu_sc as plsc

N = 10000
NP = 10240
E = 320000
HEADS = 4
F = 128
D_IN = 128

NC = 2
NS = 16
L = 16

EPT = E // NS
CH = 128
NCHUNK = (EPT + CH - 1) // CH
EPT_PAD = NCHUNK * CH
ROWS_PER_TILE = NP // NS

TCB = 1280
GRID = NP // TCB


def _proj_body(x_ref, w_ref, al_ref, ar_ref, feat_ref, el_ref, er_ref):
    y = jnp.dot(x_ref[...], w_ref[...], preferred_element_type=jnp.float32)
    for h in range(HEADS):
        fh = y[:, h * F:(h + 1) * F]
        feat_ref[h] = fh
        el_ref[h] = jnp.sum(fh * al_ref[h][None, :], axis=-1)
        er_ref[h] = jnp.sum(fh * ar_ref[h][None, :], axis=-1)


def _dense1(x, w, al, ar):
    return pl.pallas_call(
        _proj_body,
        grid=(GRID,),
        in_specs=[
            pl.BlockSpec((TCB, D_IN), lambda i: (i, 0)),
            pl.BlockSpec((D_IN, HEADS * F), lambda i: (0, 0)),
            pl.BlockSpec((HEADS, F), lambda i: (0, 0)),
            pl.BlockSpec((HEADS, F), lambda i: (0, 0)),
        ],
        out_specs=[
            pl.BlockSpec((HEADS, TCB, F), lambda i: (0, i, 0)),
            pl.BlockSpec((HEADS, TCB), lambda i: (0, i)),
            pl.BlockSpec((HEADS, TCB), lambda i: (0, i)),
        ],
        out_shape=[
            jax.ShapeDtypeStruct((HEADS, NP, F), jnp.float32),
            jax.ShapeDtypeStruct((HEADS, NP), jnp.float32),
            jax.ShapeDtypeStruct((HEADS, NP), jnp.float32),
        ],
    )(x, w, al, ar)


def _mid_body(o_ref, b_ref, w_ref, al_ref, ar_ref, feat_ref, el_ref, er_ref):
    acc = jnp.maximum(o_ref[0] + b_ref[0][None, :], 0.0)
    for h in range(1, HEADS):
        acc = acc + jnp.maximum(o_ref[h] + b_ref[h][None, :], 0.0)
    hm = acc * (1.0 / HEADS)
    y = jnp.dot(hm, w_ref[...], preferred_element_type=jnp.float32)
    for h in range(HEADS):
        fh = y[:, h * F:(h + 1) * F]
        feat_ref[h] = fh
        el_ref[h] = jnp.sum(fh * al_ref[h][None, :], axis=-1)
        er_ref[h] = jnp.sum(fh * ar_ref[h][None, :], axis=-1)


def _dense2(o1, b1r, w, al, ar):
    return pl.pallas_call(
        _mid_body,
        grid=(GRID,),
        in_specs=[
            pl.BlockSpec((HEADS, TCB, F), lambda i: (0, i, 0)),
            pl.BlockSpec((HEADS, F), lambda i: (0, 0)),
            pl.BlockSpec((F, HEADS * F), lambda i: (0, 0)),
            pl.BlockSpec((HEADS, F), lambda i: (0, 0)),
            pl.BlockSpec((HEADS, F), lambda i: (0, 0)),
        ],
        out_specs=[
            pl.BlockSpec((HEADS, TCB, F), lambda i: (0, i, 0)),
            pl.BlockSpec((HEADS, TCB), lambda i: (0, i)),
            pl.BlockSpec((HEADS, TCB), lambda i: (0, i)),
        ],
        out_shape=[
            jax.ShapeDtypeStruct((HEADS, NP, F), jnp.float32),
            jax.ShapeDtypeStruct((HEADS, NP), jnp.float32),
            jax.ShapeDtypeStruct((HEADS, NP), jnp.float32),
        ],
    )(o1, b1r, w, al, ar)


def _final_body(o_ref, b_ref, x_ref, out_ref):
    acc = o_ref[0] + b_ref[0][None, :]
    for h in range(1, HEADS):
        acc = acc + (o_ref[h] + b_ref[h][None, :])
    out_ref[...] = acc * (1.0 / HEADS) + x_ref[...]


def _dense3(o2, b2r, x):
    return pl.pallas_call(
        _final_body,
        grid=(GRID,),
        in_specs=[
            pl.BlockSpec((HEADS, TCB, F), lambda i: (0, i, 0)),
            pl.BlockSpec((HEADS, F), lambda i: (0, 0)),
            pl.BlockSpec((TCB, F), lambda i: (i, 0)),
        ],
        out_specs=pl.BlockSpec((TCB, F), lambda i: (i, 0)),
        out_shape=jax.ShapeDtypeStruct((NP, F), jnp.float32),
    )(o2, b2r, x)


def _sc_body(src_hbm, dst_hbm, feat_hbm, el_hbm, er_hbm, out_hbm,
             tab_v, tab2_v, inv_v, rows_v, scb, dcb, idx_v, ab, zb,
             out_sh, den_sh):
    c = lax.axis_index("c")
    s = lax.axis_index("s")
    lane = lax.iota(jnp.int32, L)

    def zrow(i, _):
        for k in range(F // L):
            rows_v[i, pl.ds(k * L, L)] = jnp.zeros((L,), jnp.float32)
        return 0
    lax.fori_loop(0, CH, zrow, 0)
    for k in range(CH // L):
        zb[pl.ds(k * L, L)] = jnp.zeros((L,), jnp.float32)

    for hh in range(HEADS // NC):
        h = c * (HEADS // NC) + hh
        base = h * NP

        for k in range(ROWS_PER_TILE // CH):
            pltpu.sync_copy(zb, den_sh.at[pl.ds(s * ROWS_PER_TILE + k * CH, CH)])
            pltpu.sync_copy(
                rows_v, out_sh.at[pl.ds(s * ROWS_PER_TILE + k * CH, CH)])

        pltpu.sync_copy(el_hbm.at[pl.ds(base, NP)], tab_v)
        pltpu.sync_copy(er_hbm.at[pl.ds(base, NP)], tab2_v)

        def tmax(i, m):
            m0 = jnp.maximum(m[0], tab_v[pl.ds(i * L, L)])
            m1 = jnp.maximum(m[1], tab2_v[pl.ds(i * L, L)])
            return (m0, m1)
        neg = jnp.full((L,), -1e30, jnp.float32)
        m0, m1 = lax.fori_loop(0, NP // L, tmax, (neg, neg))
        mtot = jnp.max(m0) + jnp.max(m1)
        cshift = jnp.where(mtot >= 0.0, mtot, 0.2 * mtot)
        g = jnp.full((L,), cshift, jnp.float32)

        plsc.subcore_barrier()

        def chunkA(j, _):
            pltpu.sync_copy(src_hbm.at[s].at[pl.ds(j, 1)], scb)
            pltpu.sync_copy(dst_hbm.at[s].at[pl.ds(j, 1)], dcb)
            for k in range(CH // L):
                sv = scb[0, pl.ds(k * L, L)]
                dv = dcb[0, pl.ds(k * L, L)]
                ev = plsc.load_gather(tab_v, [sv]) + plsc.load_gather(tab2_v, [dv])
                ev = jnp.where(ev >= 0.0, ev, 0.2 * ev)
                ex = jnp.exp(ev - g)
                gid = j * CH + k * L + lane
                ab[pl.ds(k * L, L)] = jnp.where(gid < EPT, ex, 0.0)
            pltpu.sync_copy(ab, den_sh.at[dcb.at[0]], add=True)
            return 0
        lax.fori_loop(0, NCHUNK, chunkA, 0)
        plsc.subcore_barrier()

        pltpu.sync_copy(den_sh, inv_v)

        def invv(i, _):
            d = inv_v[pl.ds(i * L, L)]
            inv_v[pl.ds(i * L, L)] = 1.0 / jnp.maximum(d, 1e-9)
            return 0
        lax.fori_loop(0, NP // L, invv, 0)

        def chunkB(j, _):
            pltpu.sync_copy(src_hbm.at[s].at[pl.ds(j, 1)], scb)
            pltpu.sync_copy(dst_hbm.at[s].at[pl.ds(j, 1)], dcb)
            for k in range(CH // L):
                sv = scb[0, pl.ds(k * L, L)]
                dv = dcb[0, pl.ds(k * L, L)]
                ev = plsc.load_gather(tab_v, [sv]) + plsc.load_gather(tab2_v, [dv])
                ev = jnp.where(ev >= 0.0, ev, 0.2 * ev)
                ex = jnp.exp(ev - g)
                gid = j * CH + k * L + lane
                al = ex * plsc.load_gather(inv_v, [dv])
                ab[pl.ds(k * L, L)] = jnp.where(gid < EPT, al, 0.0)
                idx_v[pl.ds(k * L, L)] = sv + base
            pltpu.sync_copy(feat_hbm.at[idx_v], rows_v)

            def scale(e, _):
                a = plsc.load_gather(ab, [jnp.full((L,), e, jnp.int32)])
                for k in range(F // L):
                    rows_v[e, pl.ds(k * L, L)] = rows_v[e, pl.ds(k * L, L)] * a
                return 0
            lax.fori_loop(0, CH, scale, 0)
            pltpu.sync_copy(rows_v, out_sh.at[dcb.at[0]], add=True)
            return 0
        lax.fori_loop(0, NCHUNK, chunkB, 0)
        plsc.subcore_barrier()

        pltpu.sync_copy(
            out_sh.at[pl.ds(s * ROWS_PER_TILE, ROWS_PER_TILE)],
            out_hbm.at[pl.ds(base + s * ROWS_PER_TILE, ROWS_PER_TILE)])
        plsc.subcore_barrier()

        if hh == 0:
            lax.fori_loop(0, CH, zrow, 0)


_sc_layer = functools.partial(
    pl.kernel,
    out_type=jax.ShapeDtypeStruct((HEADS * NP, F), jnp.float32),
    mesh=plsc.VectorSubcoreMesh(
        core_axis_name="c", subcore_axis_name="s",
        num_cores=NC, num_subcores=NS),
    compiler_params=pltpu.CompilerParams(needs_layout_passes=False),
    scratch_types=[
        pltpu.VMEM((NP,), jnp.float32),
        pltpu.VMEM((NP,), jnp.float32),
        pltpu.VMEM((NP,), jnp.float32),
        pltpu.VMEM((CH, F), jnp.float32),
        pltpu.VMEM((1, CH), jnp.int32),
        pltpu.VMEM((1, CH), jnp.int32),
        pltpu.VMEM((CH,), jnp.int32),
        pltpu.VMEM((CH,), jnp.float32),
        pltpu.VMEM((CH,), jnp.float32),
        pltpu.VMEM_SHARED((NP, F), jnp.float32),
        pltpu.VMEM_SHARED((NP,), jnp.float32),
    ],
)(_sc_body)


@jax.jit
def _run(in_feat, src3, dst3, W1, al1, ar1, b1, W2, al2, ar2, b2):
    x = jnp.pad(in_feat, ((0, NP - N), (0, 0)))
    feat1, el1, er1 = _dense1(x, W1, al1, ar1)
    o1 = _sc_layer(src3, dst3,
                   feat1.reshape(HEADS * NP, F),
                   el1.reshape(HEADS * NP), er1.reshape(HEADS * NP))
    feat2, el2, er2 = _dense2(o1.reshape(HEADS, NP, F),
                              b1.reshape(HEADS, F), W2, al2, ar2)
    o2 = _sc_layer(src3, dst3,
                   feat2.reshape(HEADS * NP, F),
                   el2.reshape(HEADS * NP), er2.reshape(HEADS * NP))
    out = _dense3(o2.reshape(HEADS, NP, F), b2.reshape(HEADS, F), x)
    return out[:N]


def kernel(in_feat, edge_index, W1, al1, ar1, b1, W2, al2, ar2, b2):
    src = edge_index[0].astype(jnp.int32).reshape(NS, EPT)
    dst = edge_index[1].astype(jnp.int32).reshape(NS, EPT)
    src3 = jnp.pad(src, ((0, 0), (0, EPT_PAD - EPT))).reshape(NS, NCHUNK, CH)
    dst3 = jnp.pad(dst, ((0, 0), (0, EPT_PAD - EPT))).reshape(NS, NCHUNK, CH)
    return _run(in_feat, src3, dst3, W1, al1, ar1, b1, W2, al2, ar2, b2)
